# baseline (speedup 1.0000x reference)
import jax
import jax.numpy as jnp
from jax import lax
from jax.experimental import pallas as pl
from jax.experimental.pallas import tpu as pltpu
from jax.experimental.pallas import tpu_sc as plsc

N = 100000
E = 3200000
D = 16
NC = 2
NS = 16
NW = NC * NS
EW = E // NW
CH = 400
NCHUNK = EW // CH
NPAIR = NCHUNK // 2
NP = 100096
RP = NP // NS
KZ = RP // CH
RZ = RP % CH


def _sc_body(x_hbm, ef_hbm, zagg_hbm, zcnt_hbm, ones_hbm,
             agg_out, cnt_out,
             isA, idA, isB, idB, rowsA, rowsB, ones_v, zv,
             agg_sh, cnt_sh, sIA, sIB, sGA, sGB, sSA, sSB):
    c = lax.axis_index("c")
    s = lax.axis_index("s")
    wid = c * NS + s
    ebase = wid * EW

    for k in range(KZ):
        pltpu.sync_copy(zagg_hbm, agg_sh.at[pl.ds(s * RP + k * CH, CH)])
    pltpu.sync_copy(zagg_hbm.at[pl.ds(0, RZ)],
                    agg_sh.at[pl.ds(s * RP + KZ * CH, RZ)])
    pltpu.sync_copy(zcnt_hbm, zv)
    for k in range(KZ):
        pltpu.sync_copy(zv, cnt_sh.at[pl.ds(s * RP + k * CH, CH)])
    pltpu.sync_copy(zv.at[pl.ds(0, RZ)], cnt_sh.at[pl.ds(s * RP + KZ * CH, RZ)])
    pltpu.sync_copy(ones_hbm, ones_v)
    plsc.subcore_barrier()

    def idx_start(chunk, is_ref, id_ref, sem):
        b = ebase + chunk * CH
        pltpu.async_copy(ef_hbm.at[pl.ds(b, CH)], is_ref, sem)
        pltpu.async_copy(ef_hbm.at[pl.ds(E + b, CH)], id_ref, sem)

    def idx_wait(is_ref, id_ref, sem):
        pltpu.make_async_copy(ef_hbm.at[pl.ds(0, CH)], is_ref, sem).wait()
        pltpu.make_async_copy(ef_hbm.at[pl.ds(0, CH)], id_ref, sem).wait()

    idx_start(0, isA, idA, sIA)
    idx_start(1, isB, idB, sIB)

    def pair(p, carry):
        na = jnp.minimum(2 * p + 2, NCHUNK - 1)
        nb = jnp.minimum(2 * p + 3, NCHUNK - 1)
        idx_wait(isA, idA, sIA)
        gA = pltpu.async_copy(x_hbm.at[isA], rowsA, sGA)
        idx_wait(isB, idB, sIB)
        gB = pltpu.async_copy(x_hbm.at[isB], rowsB, sGB)
        gA.wait()
        gB.wait()
        idx_start(na, isA, idA, sIA)
        idx_start(nb, isB, idB, sIB)
        return carry

    lax.fori_loop(0, NPAIR, pair, 0)
    idx_wait(isA, idA, sIA)
    idx_wait(isB, idB, sIB)
    plsc.subcore_barrier()

    pltpu.sync_copy(agg_sh.at[pl.ds(s * RP, RP)],
                    agg_out.at[pl.ds(c * NP + s * RP, RP)])
    for k in range(KZ):
        pltpu.sync_copy(cnt_sh.at[pl.ds(s * RP + k * CH, CH)], zv)
        pltpu.sync_copy(zv, cnt_out.at[pl.ds(c * NP + s * RP + k * CH, CH)])
    pltpu.sync_copy(cnt_sh.at[pl.ds(s * RP + KZ * CH, RZ)], zv.at[pl.ds(0, RZ)])
    pltpu.sync_copy(zv.at[pl.ds(0, RZ)],
                    cnt_out.at[pl.ds(c * NP + s * RP + KZ * CH, RZ)])


def _sc_scatter(x, ef, zagg, zcnt, ones):
    mesh = plsc.VectorSubcoreMesh(core_axis_name="c", subcore_axis_name="s")
    f = pl.kernel(
        _sc_body,
        out_type=[
            jax.ShapeDtypeStruct((NC * NP, D), jnp.float32),
            jax.ShapeDtypeStruct((NC * NP,), jnp.float32),
        ],
        mesh=mesh,
        scratch_types=[
            pltpu.VMEM((CH,), jnp.int32),
            pltpu.VMEM((CH,), jnp.int32),
            pltpu.VMEM((CH,), jnp.int32),
            pltpu.VMEM((CH,), jnp.int32),
            pltpu.VMEM((CH, D), jnp.float32),
            pltpu.VMEM((CH, D), jnp.float32),
            pltpu.VMEM((CH,), jnp.float32),
            pltpu.VMEM((CH,), jnp.float32),
            pltpu.VMEM_SHARED((NP, D), jnp.float32),
            pltpu.VMEM_SHARED((NP,), jnp.float32),
            pltpu.SemaphoreType.DMA,
            pltpu.SemaphoreType.DMA,
            pltpu.SemaphoreType.DMA,
            pltpu.SemaphoreType.DMA,
            pltpu.SemaphoreType.DMA,
            pltpu.SemaphoreType.DMA,
        ],
        compiler_params=pltpu.CompilerParams(use_tc_tiling_on_sc=False),
    )
    return f(x, ef, zagg, zcnt, ones)


BR = 10000


def _tc_body(agg_ref, cnt_ref, x_ref, wl_ref, bl_ref, wr_ref, o_ref):
    a = agg_ref[0] + agg_ref[1]
    cnt = cnt_ref[:, 0] + cnt_ref[:, 1]
    mean = a / jnp.maximum(cnt, 1.0)[:, None]
    t1 = lax.dot_general(mean, wl_ref[...], (((1,), (1,)), ((), ())),
                         preferred_element_type=jnp.float32)
    t2 = lax.dot_general(x_ref[...], wr_ref[...], (((1,), (1,)), ((), ())),
                         preferred_element_type=jnp.float32)
    o_ref[...] = t1 + t2 + bl_ref[...]


def _tc_combine(agg2, cnt2, x, W_l, b_l, W_r):
    grid = (N // BR,)
    return pl.pallas_call(
        _tc_body,
        grid=grid,
        in_specs=[
            pl.BlockSpec((NC, BR, D), lambda i: (0, i, 0)),
            pl.BlockSpec((BR, NC), lambda i: (i, 0)),
            pl.BlockSpec((BR, D), lambda i: (i, 0)),
            pl.BlockSpec((D, D), lambda i: (0, 0)),
            pl.BlockSpec((1, D), lambda i: (0, 0)),
            pl.BlockSpec((D, D), lambda i: (0, 0)),
        ],
        out_specs=pl.BlockSpec((BR, D), lambda i: (i, 0)),
        out_shape=jax.ShapeDtypeStruct((N, D), jnp.float32),
    )(agg2, cnt2, x, W_l, b_l, W_r)


@jax.jit
def kernel(x, edge_index, W_l, b_l, W_r):
    ef = edge_index.astype(jnp.int32).reshape(2 * E)
    zagg = jnp.zeros((CH, D), jnp.float32)
    zcnt = jnp.zeros((CH,), jnp.float32)
    ones = jnp.ones((CH,), jnp.float32)
    agg2, cnt2 = _sc_scatter(x, ef, zagg, zcnt, ones)
    agg2 = agg2.reshape(NC, NP, D)
    cnt2 = cnt2.reshape(NC, NP).T
    return _tc_combine(agg2, cnt2, x, W_l, b_l.reshape(1, D), W_r)


# speedup vs baseline: 1.0082x; 1.0082x over previous
import jax
import jax.numpy as jnp
from jax import lax
from jax.experimental import pallas as pl
from jax.experimental.pallas import tpu as pltpu
from jax.experimental.pallas import tpu_sc as plsc

N = 100000
E = 3200000
D = 16
NC = 2
NS = 16
NW = NC * NS
EW = E // NW
CH = 200
NCHUNK = EW // CH
NSET = 4
NQUAD = NCHUNK // NSET
NP = 100096
RP = NP // NS
ZB = 2048
KZ = RP // ZB
RZ = RP % ZB


def _sc_body(x_hbm, ei_hbm, zagg_hbm, zcnt_hbm, ones_hbm,
             agg_out, cnt_out,
             is0, id0, is1, id1, is2, id2, is3, id3,
             rows0, rows1, rows2, rows3, ones_v, zv,
             agg_sh, cnt_sh,
             sI0, sI1, sI2, sI3, sG0, sG1, sG2, sG3, sS0, sS1, sS2, sS3,
             sZ):
    c = lax.axis_index("c")
    s = lax.axis_index("s")
    wid = c * NS + s
    ebase = wid * EW
    IS = (is0, is1, is2, is3)
    ID = (id0, id1, id2, id3)
    ROWS = (rows0, rows1, rows2, rows3)
    SI = (sI0, sI1, sI2, sI3)
    SG = (sG0, sG1, sG2, sG3)
    SS = (sS0, sS1, sS2, sS3)

    zws = []
    for k in range(KZ):
        zws.append(pltpu.async_copy(
            zagg_hbm, agg_sh.at[pl.ds(s * RP + k * ZB, ZB)], sZ))
    zws.append(pltpu.async_copy(
        zagg_hbm.at[pl.ds(0, RZ)], agg_sh.at[pl.ds(s * RP + KZ * ZB, RZ)], sZ))
    pltpu.sync_copy(zcnt_hbm, zv)
    for k in range(KZ):
        zws.append(pltpu.async_copy(
            zv, cnt_sh.at[pl.ds(s * RP + k * ZB, ZB)], sZ))
    zws.append(pltpu.async_copy(
        zv.at[pl.ds(0, RZ)], cnt_sh.at[pl.ds(s * RP + KZ * ZB, RZ)], sZ))
    pltpu.sync_copy(ones_hbm, ones_v)
    for w in zws:
        w.wait()
    plsc.subcore_barrier()

    def idx_start(chunk, j, sem):
        b = ebase + chunk * CH
        pltpu.async_copy(ei_hbm.at[0, pl.ds(b, CH)], IS[j], sem)
        pltpu.async_copy(ei_hbm.at[1, pl.ds(b, CH)], ID[j], sem)

    def idx_wait(j, sem):
        pltpu.make_async_copy(ei_hbm.at[0, pl.ds(0, CH)], IS[j], sem).wait()
        pltpu.make_async_copy(ei_hbm.at[0, pl.ds(0, CH)], ID[j], sem).wait()

    for j in range(NSET):
        idx_start(j, j, SI[j])

    def quad(q, carry):
        k0 = q * NSET
        gs = []
        for j in range(NSET):
            idx_wait(j, SI[j])
            gs.append(pltpu.async_copy(x_hbm.at[IS[j]], ROWS[j], SG[j]))
        sc = []
        for j in range(NSET):
            gs[j].wait()
            sc.append(pltpu.async_copy(ROWS[j], agg_sh.at[ID[j]], SS[j], add=True))
            sc.append(pltpu.async_copy(ones_v, cnt_sh.at[ID[j]], SS[j], add=True))
        for j in range(NSET):
            sc[2 * j].wait()
            sc[2 * j + 1].wait()
            nxt = jnp.minimum(k0 + NSET + j, NCHUNK - 1)
            idx_start(nxt, j, SI[j])
        return carry

    lax.fori_loop(0, NQUAD, quad, 0)
    for j in range(NSET):
        idx_wait(j, SI[j])
    plsc.subcore_barrier()

    wb = pltpu.async_copy(agg_sh.at[pl.ds(s * RP, RP)],
                          agg_out.at[pl.ds(c * NP + s * RP, RP)], sZ)
    for k in range(KZ):
        pltpu.sync_copy(cnt_sh.at[pl.ds(s * RP + k * ZB, ZB)], zv)
        pltpu.sync_copy(zv, cnt_out.at[pl.ds(c * NP + s * RP + k * ZB, ZB)])
    pltpu.sync_copy(cnt_sh.at[pl.ds(s * RP + KZ * ZB, RZ)], zv.at[pl.ds(0, RZ)])
    pltpu.sync_copy(zv.at[pl.ds(0, RZ)],
                    cnt_out.at[pl.ds(c * NP + s * RP + KZ * ZB, RZ)])
    wb.wait()


def _sc_scatter(x, ei, zagg, zcnt, ones):
    mesh = plsc.VectorSubcoreMesh(core_axis_name="c", subcore_axis_name="s")
    f = pl.kernel(
        _sc_body,
        out_type=[
            jax.ShapeDtypeStruct((NC * NP, D), jnp.float32),
            jax.ShapeDtypeStruct((NC * NP,), jnp.float32),
        ],
        mesh=mesh,
        scratch_types=(
            [pltpu.VMEM((CH,), jnp.int32) for _ in range(2 * NSET)]
            + [pltpu.VMEM((CH, D), jnp.float32) for _ in range(NSET)]
            + [
                pltpu.VMEM((CH,), jnp.float32),
                pltpu.VMEM((ZB,), jnp.float32),
                pltpu.VMEM_SHARED((NP, D), jnp.float32),
                pltpu.VMEM_SHARED((NP,), jnp.float32),
            ]
            + [pltpu.SemaphoreType.DMA for _ in range(3 * NSET + 1)]
        ),
        compiler_params=pltpu.CompilerParams(use_tc_tiling_on_sc=False),
    )
    return f(x, ei, zagg, zcnt, ones)


BR = 10000


def _tc_body(agg_ref, cnt_ref, x_ref, wl_ref, bl_ref, wr_ref, o_ref):
    a = agg_ref[0] + agg_ref[1]
    cnt = cnt_ref[:, 0] + cnt_ref[:, 1]
    mean = a / jnp.maximum(cnt, 1.0)[:, None]
    t1 = lax.dot_general(mean, wl_ref[...], (((1,), (1,)), ((), ())),
                         preferred_element_type=jnp.float32)
    t2 = lax.dot_general(x_ref[...], wr_ref[...], (((1,), (1,)), ((), ())),
                         preferred_element_type=jnp.float32)
    o_ref[...] = t1 + t2 + bl_ref[...]


def _tc_combine(agg2, cnt2, x, W_l, b_l, W_r):
    grid = (N // BR,)
    return pl.pallas_call(
        _tc_body,
        grid=grid,
        in_specs=[
            pl.BlockSpec((NC, BR, D), lambda i: (0, i, 0)),
            pl.BlockSpec((BR, NC), lambda i: (i, 0)),
            pl.BlockSpec((BR, D), lambda i: (i, 0)),
            pl.BlockSpec((D, D), lambda i: (0, 0)),
            pl.BlockSpec((1, D), lambda i: (0, 0)),
            pl.BlockSpec((D, D), lambda i: (0, 0)),
        ],
        out_specs=pl.BlockSpec((BR, D), lambda i: (i, 0)),
        out_shape=jax.ShapeDtypeStruct((N, D), jnp.float32),
    )(agg2, cnt2, x, W_l, b_l, W_r)


@jax.jit
def kernel(x, edge_index, W_l, b_l, W_r):
    ei = edge_index.astype(jnp.int32)
    zagg = jnp.zeros((ZB, D), jnp.float32)
    zcnt = jnp.zeros((ZB,), jnp.float32)
    ones = jnp.ones((CH,), jnp.float32)
    agg2, cnt2 = _sc_scatter(x, ei, zagg, zcnt, ones)
    agg2 = agg2.reshape(NC, NP, D)
    cnt2 = cnt2.reshape(NC, NP).T
    return _tc_combine(agg2, cnt2, x, W_l, b_l.reshape(1, D), W_r)


# speedup vs baseline: 1.0889x; 1.0800x over previous
import jax
import jax.numpy as jnp
from jax import lax
from jax.experimental import pallas as pl
from jax.experimental.pallas import tpu as pltpu
from jax.experimental.pallas import tpu_sc as plsc

N = 100000
E = 3200000
D = 16
NC = 2
NS = 16
NW = NC * NS
EW = E // NW
CH = 200
NCHUNK = EW // CH
NSET = 4
NQUAD = NCHUNK // NSET
NP = 100352
RP = NP // NS
ZB = 2048
KZ = RP // ZB
RZ = RP % ZB


def _sc_body(x_hbm, ei_hbm, zagg_hbm, zcnt_hbm, ones_hbm,
             agg_out, cnt_out,
             is0, id0, is1, id1, is2, id2, is3, id3,
             rows0, rows1, rows2, rows3, ones_v, zv,
             agg_sh, cnt_sh,
             sI0, sI1, sI2, sI3, sG0, sG1, sG2, sG3, sS0, sS1, sS2, sS3,
             sZ):
    c = lax.axis_index("c")
    s = lax.axis_index("s")
    wid = c * NS + s
    ebase = wid * EW
    IS = (is0, is1, is2, is3)
    ID = (id0, id1, id2, id3)
    ROWS = (rows0, rows1, rows2, rows3)
    SI = (sI0, sI1, sI2, sI3)
    SG = (sG0, sG1, sG2, sG3)
    SS = (sS0, sS1, sS2, sS3)

    zws = []
    for k in range(KZ):
        zws.append(pltpu.async_copy(
            zagg_hbm, agg_sh.at[pl.ds(s * RP + k * ZB, ZB)], sZ))
    zws.append(pltpu.async_copy(
        zagg_hbm.at[pl.ds(0, RZ)], agg_sh.at[pl.ds(s * RP + KZ * ZB, RZ)], sZ))
    pltpu.sync_copy(zcnt_hbm, zv)
    for k in range(KZ):
        zws.append(pltpu.async_copy(
            zv, cnt_sh.at[pl.ds(s * RP + k * ZB, ZB)], sZ))
    zws.append(pltpu.async_copy(
        zv.at[pl.ds(0, RZ)], cnt_sh.at[pl.ds(s * RP + KZ * ZB, RZ)], sZ))
    pltpu.sync_copy(ones_hbm, ones_v)
    for w in zws:
        w.wait()
    plsc.subcore_barrier()

    def idx_start(chunk, j, sem):
        b = ebase + chunk * CH
        pltpu.async_copy(ei_hbm.at[0, pl.ds(b, CH)], IS[j], sem)
        pltpu.async_copy(ei_hbm.at[1, pl.ds(b, CH)], ID[j], sem)

    def idx_wait(j, sem):
        pltpu.make_async_copy(ei_hbm.at[0, pl.ds(0, CH)], IS[j], sem).wait()
        pltpu.make_async_copy(ei_hbm.at[0, pl.ds(0, CH)], ID[j], sem).wait()

    for j in range(NSET):
        idx_start(j, j, SI[j])

    def quad(q, carry):
        k0 = q * NSET
        gs = []
        for j in range(NSET):
            idx_wait(j, SI[j])
            gs.append(pltpu.async_copy(x_hbm.at[IS[j]], ROWS[j], SG[j]))
        sc = []
        for j in range(NSET):
            gs[j].wait()
            sc.append(pltpu.async_copy(ROWS[j], agg_sh.at[ID[j]], SS[j], add=True))
            sc.append(pltpu.async_copy(ones_v, cnt_sh.at[ID[j]], SS[j], add=True))
        for j in range(NSET):
            sc[2 * j].wait()
            sc[2 * j + 1].wait()
            nxt = jnp.minimum(k0 + NSET + j, NCHUNK - 1)
            idx_start(nxt, j, SI[j])
        return carry

    lax.fori_loop(0, NQUAD, quad, 0)
    for j in range(NSET):
        idx_wait(j, SI[j])
    plsc.subcore_barrier()

    wb = pltpu.async_copy(agg_sh.at[pl.ds(s * RP, RP)],
                          agg_out.at[pl.ds(c * NP + s * RP, RP)], sZ)
    for k in range(KZ):
        pltpu.sync_copy(cnt_sh.at[pl.ds(s * RP + k * ZB, ZB)], zv)
        pltpu.sync_copy(zv, cnt_out.at[pl.ds(c * NP + s * RP + k * ZB, ZB)])
    pltpu.sync_copy(cnt_sh.at[pl.ds(s * RP + KZ * ZB, RZ)], zv.at[pl.ds(0, RZ)])
    pltpu.sync_copy(zv.at[pl.ds(0, RZ)],
                    cnt_out.at[pl.ds(c * NP + s * RP + KZ * ZB, RZ)])
    wb.wait()


def _sc_scatter(x, ei, zagg, zcnt, ones):
    mesh = plsc.VectorSubcoreMesh(core_axis_name="c", subcore_axis_name="s")
    f = pl.kernel(
        _sc_body,
        out_type=[
            jax.ShapeDtypeStruct((NC * NP, D), jnp.float32),
            jax.ShapeDtypeStruct((NC * NP,), jnp.float32),
        ],
        mesh=mesh,
        scratch_types=(
            [pltpu.VMEM((CH,), jnp.int32) for _ in range(2 * NSET)]
            + [pltpu.VMEM((CH, D), jnp.float32) for _ in range(NSET)]
            + [
                pltpu.VMEM((CH,), jnp.float32),
                pltpu.VMEM((ZB,), jnp.float32),
                pltpu.VMEM_SHARED((NP, D), jnp.float32),
                pltpu.VMEM_SHARED((NP,), jnp.float32),
            ]
            + [pltpu.SemaphoreType.DMA for _ in range(3 * NSET + 1)]
        ),
        compiler_params=pltpu.CompilerParams(use_tc_tiling_on_sc=False),
    )
    return f(x, ei, zagg, zcnt, ones)


BR = 7168
GRID = NP // BR


def _tc_body(agg0_ref, agg1_ref, cnt0_ref, cnt1_ref, x_ref,
             wl_ref, bl_ref, wr_ref, o_ref):
    a = agg0_ref[...] + agg1_ref[...]
    cnt = cnt0_ref[...] + cnt1_ref[...]
    mean = a / jnp.maximum(cnt, 1.0)[:, None]
    t1 = lax.dot_general(mean, wl_ref[...], (((1,), (1,)), ((), ())),
                         preferred_element_type=jnp.float32)
    t2 = lax.dot_general(x_ref[...], wr_ref[...], (((1,), (1,)), ((), ())),
                         preferred_element_type=jnp.float32)
    o_ref[...] = t1 + t2 + bl_ref[...]


def _tc_combine(agg2, cnt2, x, W_l, b_l, W_r):
    return pl.pallas_call(
        _tc_body,
        grid=(GRID,),
        in_specs=[
            pl.BlockSpec((BR, D), lambda i: (i, 0)),
            pl.BlockSpec((BR, D), lambda i: (i + GRID, 0)),
            pl.BlockSpec((BR,), lambda i: (i,)),
            pl.BlockSpec((BR,), lambda i: (i + GRID,)),
            pl.BlockSpec((BR, D), lambda i: (i, 0)),
            pl.BlockSpec((D, D), lambda i: (0, 0)),
            pl.BlockSpec((1, D), lambda i: (0, 0)),
            pl.BlockSpec((D, D), lambda i: (0, 0)),
        ],
        out_specs=pl.BlockSpec((BR, D), lambda i: (i, 0)),
        out_shape=jax.ShapeDtypeStruct((N, D), jnp.float32),
    )(agg2, agg2, cnt2, cnt2, x, W_l, b_l, W_r)


@jax.jit
def kernel(x, edge_index, W_l, b_l, W_r):
    ei = edge_index.astype(jnp.int32)
    zagg = jnp.zeros((ZB, D), jnp.float32)
    zcnt = jnp.zeros((ZB,), jnp.float32)
    ones = jnp.ones((CH,), jnp.float32)
    agg2, cnt2 = _sc_scatter(x, ei, zagg, zcnt, ones)
    return _tc_combine(agg2, cnt2, x, W_l, b_l.reshape(1, D), W_r)


# speedup vs baseline: 1.1621x; 1.0672x over previous
import jax
import jax.numpy as jnp
from jax import lax
from jax.experimental import pallas as pl
from jax.experimental.pallas import tpu as pltpu
from jax.experimental.pallas import tpu_sc as plsc

N = 100000
E = 3200000
D = 16
NC = 2
NS = 16
NW = NC * NS
EW = E // NW
CH = 200
NCHUNK = EW // CH
NSET = 5
NQUAD = NCHUNK // NSET
NP = 100352
RP = NP // NS
ZB = 2048
KZ = RP // ZB
RZ = RP % ZB


def _sc_body(x_hbm, ei_hbm, zagg_hbm, zcnt_hbm, ones_hbm,
             agg_out, cnt_out, *scr):
    c = lax.axis_index("c")
    s = lax.axis_index("s")
    wid = c * NS + s
    ebase = wid * EW
    IS = scr[0:NSET]
    ID = scr[NSET:2 * NSET]
    ROWS = scr[2 * NSET:3 * NSET]
    ones_v = scr[3 * NSET]
    zv = scr[3 * NSET + 1]
    agg_sh = scr[3 * NSET + 2]
    cnt_sh = scr[3 * NSET + 3]
    SI = scr[3 * NSET + 4:4 * NSET + 4]
    SG = scr[4 * NSET + 4:5 * NSET + 4]
    SS = scr[5 * NSET + 4:6 * NSET + 4]
    sZ = scr[6 * NSET + 4]

    zws = []
    for k in range(KZ):
        zws.append(pltpu.async_copy(
            zagg_hbm, agg_sh.at[pl.ds(s * RP + k * ZB, ZB)], sZ))
    zws.append(pltpu.async_copy(
        zagg_hbm.at[pl.ds(0, RZ)], agg_sh.at[pl.ds(s * RP + KZ * ZB, RZ)], sZ))
    pltpu.sync_copy(zcnt_hbm, zv)
    for k in range(KZ):
        zws.append(pltpu.async_copy(
            zv, cnt_sh.at[pl.ds(s * RP + k * ZB, ZB)], sZ))
    zws.append(pltpu.async_copy(
        zv.at[pl.ds(0, RZ)], cnt_sh.at[pl.ds(s * RP + KZ * ZB, RZ)], sZ))
    pltpu.sync_copy(ones_hbm, ones_v)
    for w in zws:
        w.wait()
    plsc.subcore_barrier()

    def idx_start(chunk, j, sem):
        b = ebase + chunk * CH
        pltpu.async_copy(ei_hbm.at[0, pl.ds(b, CH)], IS[j], sem)
        pltpu.async_copy(ei_hbm.at[1, pl.ds(b, CH)], ID[j], sem)

    def idx_wait(j, sem):
        pltpu.make_async_copy(ei_hbm.at[0, pl.ds(0, CH)], IS[j], sem).wait()
        pltpu.make_async_copy(ei_hbm.at[0, pl.ds(0, CH)], ID[j], sem).wait()

    for j in range(NSET):
        idx_start(j, j, SI[j])

    def quad(q, carry):
        k0 = q * NSET
        gs = []
        for j in range(NSET):
            idx_wait(j, SI[j])
            gs.append(pltpu.async_copy(x_hbm.at[IS[j]], ROWS[j], SG[j]))
        sc = []
        for j in range(NSET):
            gs[j].wait()
            sc.append(pltpu.async_copy(ROWS[j], agg_sh.at[ID[j]], SS[j], add=True))
            sc.append(pltpu.async_copy(ones_v, cnt_sh.at[ID[j]], SS[j], add=True))
        for j in range(NSET):
            sc[2 * j].wait()
            sc[2 * j + 1].wait()
            nxt = jnp.minimum(k0 + NSET + j, NCHUNK - 1)
            idx_start(nxt, j, SI[j])
        return carry

    lax.fori_loop(0, NQUAD, quad, 0)
    for j in range(NSET):
        idx_wait(j, SI[j])
    plsc.subcore_barrier()

    wb = pltpu.async_copy(agg_sh.at[pl.ds(s * RP, RP)],
                          agg_out.at[pl.ds(c * NP + s * RP, RP)], sZ)
    for k in range(KZ):
        pltpu.sync_copy(cnt_sh.at[pl.ds(s * RP + k * ZB, ZB)], zv)
        pltpu.sync_copy(zv, cnt_out.at[pl.ds(c * NP + s * RP + k * ZB, ZB)])
    pltpu.sync_copy(cnt_sh.at[pl.ds(s * RP + KZ * ZB, RZ)], zv.at[pl.ds(0, RZ)])
    pltpu.sync_copy(zv.at[pl.ds(0, RZ)],
                    cnt_out.at[pl.ds(c * NP + s * RP + KZ * ZB, RZ)])
    wb.wait()


def _sc_scatter(x, ei, zagg, zcnt, ones):
    mesh = plsc.VectorSubcoreMesh(core_axis_name="c", subcore_axis_name="s")
    f = pl.kernel(
        _sc_body,
        out_type=[
            jax.ShapeDtypeStruct((NC * NP, D), jnp.float32),
            jax.ShapeDtypeStruct((NC * NP,), jnp.float32),
        ],
        mesh=mesh,
        scratch_types=(
            [pltpu.VMEM((CH,), jnp.int32) for _ in range(2 * NSET)]
            + [pltpu.VMEM((CH, D), jnp.float32) for _ in range(NSET)]
            + [
                pltpu.VMEM((CH,), jnp.float32),
                pltpu.VMEM((ZB,), jnp.float32),
                pltpu.VMEM_SHARED((NP, D), jnp.float32),
                pltpu.VMEM_SHARED((NP,), jnp.float32),
            ]
            + [pltpu.SemaphoreType.DMA for _ in range(3 * NSET + 1)]
        ),
        compiler_params=pltpu.CompilerParams(use_tc_tiling_on_sc=False),
    )
    return f(x, ei, zagg, zcnt, ones)


BR = 7168
GRID = NP // BR


def _tc_body(agg0_ref, agg1_ref, cnt0_ref, cnt1_ref, x_ref,
             wl_ref, bl_ref, wr_ref, o_ref):
    a = agg0_ref[...] + agg1_ref[...]
    cnt = cnt0_ref[...] + cnt1_ref[...]
    mean = a / jnp.maximum(cnt, 1.0)[:, None]
    t1 = lax.dot_general(mean, wl_ref[...], (((1,), (1,)), ((), ())),
                         preferred_element_type=jnp.float32)
    t2 = lax.dot_general(x_ref[...], wr_ref[...], (((1,), (1,)), ((), ())),
                         preferred_element_type=jnp.float32)
    o_ref[...] = t1 + t2 + bl_ref[...]


def _tc_combine(agg2, cnt2, x, W_l, b_l, W_r):
    return pl.pallas_call(
        _tc_body,
        grid=(GRID,),
        in_specs=[
            pl.BlockSpec((BR, D), lambda i: (i, 0)),
            pl.BlockSpec((BR, D), lambda i: (i + GRID, 0)),
            pl.BlockSpec((BR,), lambda i: (i,)),
            pl.BlockSpec((BR,), lambda i: (i + GRID,)),
            pl.BlockSpec((BR, D), lambda i: (i, 0)),
            pl.BlockSpec((D, D), lambda i: (0, 0)),
            pl.BlockSpec((1, D), lambda i: (0, 0)),
            pl.BlockSpec((D, D), lambda i: (0, 0)),
        ],
        out_specs=pl.BlockSpec((BR, D), lambda i: (i, 0)),
        out_shape=jax.ShapeDtypeStruct((N, D), jnp.float32),
    )(agg2, agg2, cnt2, cnt2, x, W_l, b_l, W_r)


@jax.jit
def kernel(x, edge_index, W_l, b_l, W_r):
    ei = edge_index.astype(jnp.int32)
    zagg = jnp.zeros((ZB, D), jnp.float32)
    zcnt = jnp.zeros((ZB,), jnp.float32)
    ones = jnp.ones((CH,), jnp.float32)
    agg2, cnt2 = _sc_scatter(x, ei, zagg, zcnt, ones)
    return _tc_combine(agg2, cnt2, x, W_l, b_l.reshape(1, D), W_r)
